# baseline (speedup 1.0000x reference)
import functools

import jax
import jax.numpy as jnp
from jax import lax
from jax.experimental import pallas as pl
from jax.experimental.pallas import tpu as pltpu
from jax.experimental.pallas import tpu_sc as plsc

_GAMMA = 0.01
_SCALE = 3.1622776601683795

_B, _G, _S, _L = 64, 2048, 64, 4
_NBH = 2
_NGT = 16
_BL = _B // _NBH
_GC = _G // _NGT
_NQ = 2


def _sc_clause(x_hbm, idx_hbm, out_hbm, xloc, iref, out_v, body_v):
    wid = lax.axis_index("s") * 2 + lax.axis_index("c")
    bh = wid // _NGT
    gt = wid % _NGT

    pltpu.sync_copy(x_hbm.at[pl.ds(bh * (_G * _BL), _G * _BL)], xloc)
    pltpu.sync_copy(idx_hbm.at[gt], iref)

    def g_body(g, _):
        p0 = g * (_S * _L)

        def t_body(t, st):
            m0, m1 = st
            iv = iref[pl.ds(p0 + t * 16, 16)]
            rows = []
            for k in range(16):
                base = iv[k] * 32
                rows.append((xloc[pl.ds(base, 16)],
                             xloc[pl.ds(base + 16, 16)]))
            for u in range(4):
                r = rows[u * _L:(u + 1) * _L]
                a0 = (r[0][0] * r[1][0]) * (r[2][0] * r[3][0])
                a1 = (r[0][1] * r[1][1]) * (r[2][1] * r[3][1])
                body_v[pl.ds((t * 4 + u) * 32, 16)] = a0
                body_v[pl.ds((t * 4 + u) * 32 + 16, 16)] = a1
                m0 = jnp.maximum(m0, a0)
                m1 = jnp.maximum(m1, a1)
            return m0, m1

        neg = jnp.full((16,), -1e30, jnp.float32)
        m0, m1 = plsc.parallel_loop(0, _S // 4, 1, unroll=1,
                                    carry=(neg, neg))(
            lambda t, st: t_body(t, st))

        def s2_body(s, st):
            s0, s1 = st
            e0 = jnp.exp(body_v[pl.ds(s * 32, 16)] - m0)
            e1 = jnp.exp(body_v[pl.ds(s * 32 + 16, 16)] - m1)
            return s0 + e0, s1 + e1

        zero = jnp.zeros((16,), jnp.float32)
        s0, s1 = lax.fori_loop(0, _S, s2_body, (zero, zero))
        for q, (mq, sq) in enumerate(((m0, s0), (m1, s1))):
            y = (sq.view(jnp.int32).astype(jnp.float32)
                 * jnp.float32(8.262958e-08)
                 - jnp.float32(88.029691931))
            y = y + sq * jnp.exp(-y) - 1.0
            y = y + sq * jnp.exp(-y) - 1.0
            out_v[pl.ds(g * 32 + q * 16, 16)] = _GAMMA * (mq + y)
        return 0

    lax.fori_loop(0, _GC, g_body, 0)

    pltpu.sync_copy(out_v, out_hbm.at[bh, gt])


_sc_call = functools.partial(
    pl.kernel,
    out_type=jax.ShapeDtypeStruct((_NBH, _NGT, _GC * _NQ * 16), jnp.float32),
    mesh=plsc.VectorSubcoreMesh(core_axis_name="c", subcore_axis_name="s"),
    compiler_params=pltpu.CompilerParams(needs_layout_passes=False),
    scratch_types=[
        pltpu.VMEM((_G * _BL,), jnp.float32),
        pltpu.VMEM((_GC * _S * _L,), jnp.int32),
        pltpu.VMEM((_GC * _NQ * 16,), jnp.float32),
        pltpu.VMEM((_S * _NQ * 16,), jnp.float32),
    ],
)(_sc_clause)


def kernel(x, I):
    xr = (x * jnp.float32(_SCALE)).reshape(_NBH, _BL, _G)
    xr = xr.transpose(0, 2, 1)
    idx = I[0].reshape(_NGT, _GC * _S * _L)
    out4 = _sc_call(xr.reshape(-1), idx)
    out4 = out4.reshape(_NBH, _NGT, _GC, _NQ, 16)
    return jnp.transpose(out4, (0, 3, 4, 1, 2)).reshape(_B, _G)


# speedup vs baseline: 2.2818x; 2.2818x over previous
import functools

import jax
import jax.numpy as jnp
from jax import lax
from jax.experimental import pallas as pl
from jax.experimental.pallas import tpu as pltpu
from jax.experimental.pallas import tpu_sc as plsc

_GAMMA = 0.01
_SCALE = 3.1622776601683795

_B, _G, _S, _L = 64, 2048, 64, 4
_NBH = 2
_NGT = 16
_BL = _B // _NBH
_GC = _G // _NGT
_NGQ = _GC // 16
_BB = 8
_NBB = _BL // _BB


def _sc_clause(x_hbm, idx_hbm, out_hbm,
               xloc, idxv, out_v, body_v):
    wid = lax.axis_index("s") * 2 + lax.axis_index("c")
    bh = wid // _NGT
    gt = wid % _NGT

    pltpu.sync_copy(x_hbm.at[pl.ds(bh * (_BL * _G), _BL * _G)], xloc)
    pltpu.sync_copy(idx_hbm.at[gt], idxv)

    for gq in range(_NGQ):
        c0 = gq * 16

        def bblk_body(bblk, _, c0=c0):
            b0 = bblk * _BB
            xrefs = [xloc.at[pl.ds((b0 + j) * _G, _G)] for j in range(_BB)]

            def s_body(s, st):
                i0 = idxv[s, 0, pl.ds(c0, 16)]
                i1 = idxv[s, 1, pl.ds(c0, 16)]
                i2 = idxv[s, 2, pl.ds(c0, 16)]
                i3 = idxv[s, 3, pl.ds(c0, 16)]
                vals = [[plsc.load_gather(xrefs[j], [i])
                         for i in (i0, i1, i2, i3)] for j in range(_BB)]
                prods = [(v[0] * v[1]) * (v[2] * v[3]) for v in vals]
                new_m = []
                for j in range(_BB):
                    body_v[s, pl.ds(j * 16, 16)] = prods[j]
                    new_m.append(jnp.maximum(st[j], prods[j]))
                return tuple(new_m)

            init = (jnp.full((16,), -1e30, jnp.float32),) * _BB
            ms = plsc.parallel_loop(0, _S, 1, unroll=2, carry=init)(
                lambda s, st: s_body(s, st))

            def s2_body(s, st):
                new_s = []
                for j in range(_BB):
                    e = jnp.exp(body_v[s, pl.ds(j * 16, 16)] - ms[j])
                    new_s.append(st[j] + e)
                return tuple(new_s)

            ss = lax.fori_loop(0, _S, s2_body,
                               (jnp.zeros((16,), jnp.float32),) * _BB)
            for j in range(_BB):
                sv = ss[j]
                y = (sv.view(jnp.int32).astype(jnp.float32)
                     * jnp.float32(8.262958e-08)
                     - jnp.float32(88.029691931))
                y = y + sv * jnp.exp(-y) - 1.0
                y = y + sv * jnp.exp(-y) - 1.0
                out_v[b0 + j, pl.ds(c0, 16)] = _GAMMA * (ms[j] + y)
            return 0

        lax.fori_loop(0, _NBB, bblk_body, 0)

    pltpu.sync_copy(
        out_v, out_hbm.at[pl.ds(bh * _BL, _BL), pl.ds(gt * _GC, _GC)])


_sc_call = functools.partial(
    pl.kernel,
    out_type=jax.ShapeDtypeStruct((_B, _G), jnp.float32),
    mesh=plsc.VectorSubcoreMesh(core_axis_name="c", subcore_axis_name="s"),
    compiler_params=pltpu.CompilerParams(needs_layout_passes=False),
    scratch_types=[
        pltpu.VMEM((_BL * _G,), jnp.float32),
        pltpu.VMEM((_S, _L, _GC), jnp.int32),
        pltpu.VMEM((_BL, _GC), jnp.float32),
        pltpu.VMEM((_S, _BB * 16), jnp.float32),
    ],
)(_sc_clause)


def kernel(x, I):
    idx = jnp.transpose(I[0], (1, 2, 0))
    idx = idx.reshape(_S, _L, _NGT, _GC).transpose(2, 0, 1, 3)
    xs = (x * jnp.float32(_SCALE)).reshape(-1)
    return _sc_call(xs, idx)
